# baseline (speedup 1.0000x reference)
import functools

import jax
import jax.numpy as jnp
from jax import lax
from jax.experimental import pallas as pl
from jax.experimental.pallas import tpu as pltpu
from jax.experimental.pallas import tpu_sc as plsc

VOCAB = 1000
D_MODEL = 128
NUM_CORES = 2
NUM_SUBCORES = 16
NW = NUM_CORES * NUM_SUBCORES
TOKENS = 1024 * 50
BPW = TOKENS // NW
CHUNK = 64
NCHUNK = BPW // CHUNK


def _proj_table_kernel(emb_ref, w_ref, b_ref, p_ref):
    p_ref[...] = (
        jnp.dot(emb_ref[...], w_ref[...], preferred_element_type=jnp.float32)
        + b_ref[...]
    )


def _make_proj_table(emb_table, W, b):
    return pl.pallas_call(
        _proj_table_kernel,
        out_shape=jax.ShapeDtypeStruct((VOCAB, VOCAB), jnp.float32),
    )(emb_table, W, b.reshape(1, VOCAB))


@functools.cache
def _make_gather_rows():
    mesh = plsc.VectorSubcoreMesh(core_axis_name="c", subcore_axis_name="s")

    @functools.partial(
        pl.kernel,
        mesh=mesh,
        out_type=jax.ShapeDtypeStruct((TOKENS, VOCAB), jnp.float32),
        scratch_types=[
            pltpu.VMEM((NCHUNK, CHUNK), jnp.int32),
            pltpu.VMEM((CHUNK, VOCAB), jnp.float32),
            pltpu.SemaphoreType.DMA,
        ],
        compiler_params=pltpu.CompilerParams(use_tc_tiling_on_sc=False),
    )
    def _gather_rows(p_hbm, idx_hbm, out_hbm, idx_v, rows_v, sem):
        wid = lax.axis_index("s") * NUM_CORES + lax.axis_index("c")
        base = wid * BPW
        pltpu.sync_copy(idx_hbm.at[wid], idx_v)

        def body(c, carry):
            pltpu.async_copy(p_hbm.at[idx_v.at[c]], rows_v, sem).wait()
            pltpu.sync_copy(rows_v, out_hbm.at[pl.ds(base + c * CHUNK, CHUNK)])
            return carry

        lax.fori_loop(0, NCHUNK, body, 0)

    return _gather_rows


def kernel(x, emb_table, W, b):
    p = _make_proj_table(emb_table, W, b)
    idx = x.astype(jnp.int32).reshape(NW, NCHUNK, CHUNK)
    out = _make_gather_rows()(p, idx)
    return out.reshape(x.shape[0], x.shape[1], VOCAB)


# speedup vs baseline: 1.0148x; 1.0148x over previous
import functools

import jax
import jax.numpy as jnp
from jax import lax
from jax.experimental import pallas as pl
from jax.experimental.pallas import tpu as pltpu
from jax.experimental.pallas import tpu_sc as plsc

VOCAB = 1000
D_MODEL = 128
NUM_CORES = 2
NUM_SUBCORES = 16
NW = NUM_CORES * NUM_SUBCORES
TOKENS = 1024 * 50
BPW = TOKENS // NW
CHUNK = 50
NCHUNK = BPW // CHUNK


def _proj_table_kernel(emb_ref, w_ref, b_ref, p_ref):
    p_ref[...] = (
        jnp.dot(emb_ref[...], w_ref[...], preferred_element_type=jnp.float32)
        + b_ref[...]
    )


def _make_proj_table(emb_table, W, b):
    return pl.pallas_call(
        _proj_table_kernel,
        out_shape=jax.ShapeDtypeStruct((VOCAB, VOCAB), jnp.float32),
    )(emb_table, W, b.reshape(1, VOCAB))


@functools.cache
def _make_gather_rows():
    mesh = plsc.VectorSubcoreMesh(core_axis_name="c", subcore_axis_name="s")

    @functools.partial(
        pl.kernel,
        mesh=mesh,
        out_type=jax.ShapeDtypeStruct((TOKENS, VOCAB), jnp.float32),
        scratch_types=[
            pltpu.VMEM((NCHUNK, CHUNK), jnp.int32),
            pltpu.VMEM((CHUNK, VOCAB), jnp.float32),
            pltpu.VMEM((CHUNK, VOCAB), jnp.float32),
            pltpu.SemaphoreType.DMA,
            pltpu.SemaphoreType.DMA,
            pltpu.SemaphoreType.DMA,
            pltpu.SemaphoreType.DMA,
        ],
        compiler_params=pltpu.CompilerParams(use_tc_tiling_on_sc=False),
    )
    def _gather_rows(p_hbm, idx_hbm, out_hbm, idx_v, rows_a, rows_b,
                     sg_a, sg_b, ss_a, ss_b):
        wid = lax.axis_index("s") * NUM_CORES + lax.axis_index("c")
        base = wid * BPW
        pltpu.sync_copy(idx_hbm.at[wid], idx_v)

        bufs = (rows_a, rows_b)
        gsems = (sg_a, sg_b)
        ssems = (ss_a, ss_b)

        def gather_src(c):
            return p_hbm.at[idx_v.at[c]]

        def out_dst(c):
            return out_hbm.at[pl.ds(base + c * CHUNK, CHUNK)]

        pltpu.async_copy(gather_src(0), bufs[0], gsems[0])

        def body(c0, carry):
            for p in range(2):
                c = c0 + p
                buf, gs, ss = bufs[p], gsems[p], ssems[p]
                other = 1 - p
                pltpu.make_async_copy(gather_src(c), buf, gs).wait()
                nxt = c + 1

                @pl.when(nxt < NCHUNK)
                def _():
                    @pl.when(c >= 1)
                    def _():
                        pltpu.make_async_copy(
                            bufs[other], out_dst(c - 1), ssems[other]
                        ).wait()

                    pltpu.async_copy(gather_src(nxt), bufs[other], gsems[other])

                pltpu.async_copy(buf, out_dst(c), ss)
            return carry

        lax.fori_loop(0, NCHUNK // 2, lambda i, c: body(i * 2, c), 0)

        pltpu.make_async_copy(bufs[0], out_dst(NCHUNK - 2), ssems[0]).wait()
        pltpu.make_async_copy(bufs[1], out_dst(NCHUNK - 1), ssems[1]).wait()

    return _gather_rows


def kernel(x, emb_table, W, b):
    p = _make_proj_table(emb_table, W, b)
    idx = x.astype(jnp.int32).reshape(NW, NCHUNK, CHUNK)
    out = _make_gather_rows()(p, idx)
    return out.reshape(x.shape[0], x.shape[1], VOCAB)


# speedup vs baseline: 1.7076x; 1.6828x over previous
import functools

import jax
import jax.numpy as jnp
from jax import lax
from jax.experimental import pallas as pl
from jax.experimental.pallas import tpu as pltpu
from jax.experimental.pallas import tpu_sc as plsc

VOCAB = 1000
D_MODEL = 128
BATCH = 1024
SEQ = 50
NUM_CORES = 2
NUM_SUBCORES = 16
NW = NUM_CORES * NUM_SUBCORES
BPW = BATCH // NW
SEQPAD = 64


def _proj_table_kernel(emb_ref, w_ref, b_ref, p_ref):
    p_ref[...] = (
        jnp.dot(emb_ref[...], w_ref[...], preferred_element_type=jnp.float32)
        + b_ref[...]
    )


def _make_proj_table(emb_table, W, b):
    return pl.pallas_call(
        _proj_table_kernel,
        out_shape=jax.ShapeDtypeStruct((VOCAB, VOCAB), jnp.float32),
    )(emb_table, W, b.reshape(1, VOCAB))


@functools.cache
def _make_gather_rows():
    mesh = plsc.VectorSubcoreMesh(core_axis_name="c", subcore_axis_name="s")

    @functools.partial(
        pl.kernel,
        mesh=mesh,
        out_type=jax.ShapeDtypeStruct((BATCH, SEQ, VOCAB), jnp.float32),
        scratch_types=[
            pltpu.VMEM((BPW * SEQPAD,), jnp.int32),
            pltpu.VMEM((SEQ, VOCAB), jnp.float32),
            pltpu.VMEM((SEQ, VOCAB), jnp.float32),
            pltpu.SemaphoreType.DMA,
            pltpu.SemaphoreType.DMA,
            pltpu.SemaphoreType.DMA,
            pltpu.SemaphoreType.DMA,
        ],
    )
    def _gather_rows(p_hbm, idx_hbm, out_hbm, idx_v, buf_a, buf_b,
                     sg_a, sg_b, ss_a, ss_b):
        wid = lax.axis_index("s") * NUM_CORES + lax.axis_index("c")
        base = wid * BPW
        pltpu.sync_copy(idx_hbm.at[pl.ds(base * SEQPAD, BPW * SEQPAD)], idx_v)

        bufs = (buf_a, buf_b)
        gsems = (sg_a, sg_b)
        ssems = (ss_a, ss_b)

        def fire_rows(bi, buf, gs):
            for g in range(SEQPAD // 16):
                n = min(16, SEQ - g * 16)
                if n <= 0:
                    break
                v = idx_v[pl.ds(bi * SEQPAD + g * 16, 16)]
                for j in range(n):
                    pltpu.async_copy(p_hbm.at[v[j]], buf.at[g * 16 + j], gs)

        def wait_rows(buf, gs):
            def w(i, carry):
                pltpu.make_async_copy(p_hbm.at[0], buf.at[0], gs).wait()
                return carry

            lax.fori_loop(0, SEQ, w, 0)

        fire_rows(0, bufs[0], gsems[0])

        def body(nb0, carry):
            for p in range(2):
                nb = nb0 + p
                buf, gs, ss = bufs[p], gsems[p], ssems[p]
                other = 1 - p
                wait_rows(buf, gs)
                nxt = nb + 1

                @pl.when(nxt < BPW)
                def _():
                    @pl.when(nb >= 1)
                    def _():
                        pltpu.make_async_copy(
                            bufs[other], out_hbm.at[base + nb - 1],
                            ssems[other],
                        ).wait()

                    fire_rows(nxt, bufs[other], gsems[other])

                pltpu.async_copy(buf, out_hbm.at[base + nb], ss)
            return carry

        lax.fori_loop(0, BPW // 2, lambda i, c: body(i * 2, c), 0)

        pltpu.make_async_copy(bufs[0], out_hbm.at[base + BPW - 2], ssems[0]).wait()
        pltpu.make_async_copy(bufs[1], out_hbm.at[base + BPW - 1], ssems[1]).wait()

    return _gather_rows


def kernel(x, emb_table, W, b):
    p = _make_proj_table(emb_table, W, b)
    idx = jnp.pad(x.astype(jnp.int32), ((0, 0), (0, SEQPAD - SEQ)))
    return _make_gather_rows()(p, idx.reshape(BATCH * SEQPAD))


# speedup vs baseline: 4.2503x; 2.4890x over previous
import functools

import jax
import jax.numpy as jnp
from jax import lax
from jax.experimental import pallas as pl
from jax.experimental.pallas import tpu as pltpu
from jax.experimental.pallas import tpu_sc as plsc

VOCAB = 1000
D_MODEL = 128
BATCH = 1024
SEQ = 50
NUM_CORES = 2
NUM_SUBCORES = 16
NW = NUM_CORES * NUM_SUBCORES
TOKENS = BATCH * SEQ
TPW = TOKENS // NW
CHUNK = 80
NCHUNK = TPW // CHUNK


@functools.cache
def _make_gather_rows():
    mesh = plsc.VectorSubcoreMesh(core_axis_name="c", subcore_axis_name="s")

    @functools.partial(
        pl.kernel,
        mesh=mesh,
        out_type=jax.ShapeDtypeStruct((TOKENS, D_MODEL), jnp.float32),
        scratch_types=[
            pltpu.VMEM((TPW,), jnp.int32),
            pltpu.VMEM((CHUNK, D_MODEL), jnp.float32),
            pltpu.VMEM((CHUNK, D_MODEL), jnp.float32),
            pltpu.SemaphoreType.DMA,
            pltpu.SemaphoreType.DMA,
            pltpu.SemaphoreType.DMA,
            pltpu.SemaphoreType.DMA,
        ],
    )
    def _gather_rows(tab_hbm, idx_hbm, out_hbm, idx_v, buf_a, buf_b,
                     sg_a, sg_b, ss_a, ss_b):
        wid = lax.axis_index("s") * NUM_CORES + lax.axis_index("c")
        base = wid * TPW
        pltpu.sync_copy(idx_hbm.at[pl.ds(base, TPW)], idx_v)

        bufs = (buf_a, buf_b)
        gsems = (sg_a, sg_b)
        ssems = (ss_a, ss_b)

        def gather_src(c):
            return tab_hbm.at[idx_v.at[pl.ds(c * CHUNK, CHUNK)]]

        def out_dst(c):
            return out_hbm.at[pl.ds(base + c * CHUNK, CHUNK)]

        pltpu.async_copy(gather_src(0), bufs[0], gsems[0])

        def body(c0, carry):
            for p in range(2):
                c = c0 + p
                buf, gs, ss = bufs[p], gsems[p], ssems[p]
                other = 1 - p
                pltpu.make_async_copy(gather_src(c), buf, gs).wait()
                nxt = c + 1

                @pl.when(nxt < NCHUNK)
                def _():
                    @pl.when(c >= 1)
                    def _():
                        pltpu.make_async_copy(
                            bufs[other], out_dst(c - 1), ssems[other]
                        ).wait()

                    pltpu.async_copy(gather_src(nxt), bufs[other], gsems[other])

                pltpu.async_copy(buf, out_dst(c), ss)
            return carry

        lax.fori_loop(0, NCHUNK // 2, lambda i, c: body(i * 2, c), 0)

        pltpu.make_async_copy(bufs[0], out_dst(NCHUNK - 2), ssems[0]).wait()
        pltpu.make_async_copy(bufs[1], out_dst(NCHUNK - 1), ssems[1]).wait()

    return _gather_rows


def _proj_kernel(wt_ref, e_ref, b_ref, o_ref):
    o_ref[0] = (
        lax.dot_general(
            wt_ref[...], e_ref[0],
            (((1,), (0,)), ((), ())),
            preferred_element_type=jnp.float32,
        )
        + b_ref[...]
    )


def _project(wt, emb_t, b_col):
    return pl.pallas_call(
        _proj_kernel,
        grid=(SEQ,),
        in_specs=[
            pl.BlockSpec((VOCAB, D_MODEL), lambda s: (0, 0)),
            pl.BlockSpec((1, D_MODEL, BATCH), lambda s: (s, 0, 0)),
            pl.BlockSpec((VOCAB, 1), lambda s: (0, 0)),
        ],
        out_specs=pl.BlockSpec((1, VOCAB, BATCH), lambda s: (s, 0, 0)),
        out_shape=jax.ShapeDtypeStruct((SEQ, VOCAB, BATCH), jnp.float32),
    )(wt, emb_t, b_col)


def kernel(x, emb_table, W, b):
    idx = x.astype(jnp.int32).T.reshape(TOKENS)
    emb_g = _make_gather_rows()(emb_table, idx)
    emb_t = emb_g.reshape(SEQ, BATCH, D_MODEL).swapaxes(1, 2)
    t = _project(W.T, emb_t, b[:, None])
    return jnp.transpose(t, (2, 0, 1))
